# baseline (speedup 1.0000x reference)
import jax
import jax.numpy as jnp
from jax import lax
from jax.experimental import pallas as pl
from jax.experimental.pallas import tpu as pltpu
from jax.experimental.pallas import tpu_sc as plsc

N = 10000
D = 16
NC, NS = 2, 16
NW = NC * NS
N_PAD = 10112
RPT = N_PAD // NS
E = 320000
EPT = 10240
E_PAD = NW * EPT
KJ = EPT // 128
CH = 10
NCHUNK = KJ // CH
HALF = NCHUNK // 2

_MESH = plsc.VectorSubcoreMesh(core_axis_name="c", subcore_axis_name="s")
_SC_PARAMS = pltpu.CompilerParams(use_tc_tiling_on_sc=False)


def _fire_gathers(vals_ref, sidx_v, buf, gsem, c):
    return [
        pltpu.async_copy(vals_ref.at[sidx_v.at[c * CH + j]],
                         buf.at[pl.ds(j * 128, 128)], gsem)
        for j in range(CH)
    ]


def _wait_gathers(vals_ref, sidx_v, buf, gsem):
    for j in range(CH):
        pltpu.make_async_copy(vals_ref.at[sidx_v.at[j]],
                              buf.at[pl.ds(j * 128, 128)], gsem).wait()


def _scatter_chunk(acc_sh, didx_v, buf, ssem, c):
    copies = [
        pltpu.async_copy(buf.at[pl.ds(j * 128, 128)],
                         acc_sh.at[didx_v.at[c * CH + j]], ssem, add=True)
        for j in range(CH)
    ]
    for cp in copies:
        cp.wait()


def _agg_pipeline(vals_ref, sidx_v, didx_v, rows_a, rows_b, acc_sh, gsem, ssem):
    _fire_gathers(vals_ref, sidx_v, rows_a, gsem, 0)

    def it(i, carry):
        c0 = 2 * i
        _wait_gathers(vals_ref, sidx_v, rows_a, gsem)
        _fire_gathers(vals_ref, sidx_v, rows_b, gsem, c0 + 1)
        _scatter_chunk(acc_sh, didx_v, rows_a, ssem, c0)
        _wait_gathers(vals_ref, sidx_v, rows_b, gsem)
        _fire_gathers(vals_ref, sidx_v, rows_a, gsem, c0 + 2)
        _scatter_chunk(acc_sh, didx_v, rows_b, ssem, c0 + 1)
        return carry

    lax.fori_loop(0, HALF - 1, it, 0)
    c0 = NCHUNK - 2
    _wait_gathers(vals_ref, sidx_v, rows_a, gsem)
    _fire_gathers(vals_ref, sidx_v, rows_b, gsem, c0 + 1)
    _scatter_chunk(acc_sh, didx_v, rows_a, ssem, c0)
    _wait_gathers(vals_ref, sidx_v, rows_b, gsem)
    _scatter_chunk(acc_sh, didx_v, rows_b, ssem, c0 + 1)


def _sc_agg_body(vals_hbm, epad_hbm, zeros_hbm, out_hbm,
                 sidx_v, didx_v, rows_a, rows_b, acc_sh, vals_sh, gsem, ssem):
    cid = lax.axis_index("c")
    sid = lax.axis_index("s")
    wid = sid * NC + cid
    pltpu.sync_copy(epad_hbm.at[0, wid], sidx_v)
    pltpu.sync_copy(epad_hbm.at[1, wid], didx_v)
    pltpu.sync_copy(vals_hbm.at[pl.ds(sid * RPT, RPT)],
                    vals_sh.at[pl.ds(sid * RPT, RPT)])
    pltpu.sync_copy(zeros_hbm.at[pl.ds(sid * RPT, RPT)],
                    acc_sh.at[pl.ds(sid * RPT, RPT)])
    plsc.subcore_barrier()
    _agg_pipeline(vals_sh, sidx_v, didx_v, rows_a, rows_b, acc_sh, gsem, ssem)
    plsc.subcore_barrier()
    pltpu.sync_copy(acc_sh.at[pl.ds(sid * RPT, RPT)],
                    out_hbm.at[cid, pl.ds(sid * RPT, RPT)])


_sc_agg = pl.kernel(
    _sc_agg_body,
    out_type=jax.ShapeDtypeStruct((NC, N_PAD, D), jnp.float32),
    mesh=_MESH,
    scratch_types=[
        pltpu.VMEM((KJ, 128), jnp.int32),
        pltpu.VMEM((KJ, 128), jnp.int32),
        pltpu.VMEM((CH * 128, D), jnp.float32),
        pltpu.VMEM((CH * 128, D), jnp.float32),
        pltpu.VMEM_SHARED((N_PAD, D), jnp.float32),
        pltpu.VMEM_SHARED((N_PAD, D), jnp.float32),
        pltpu.SemaphoreType.DMA,
        pltpu.SemaphoreType.DMA,
    ],
    compiler_params=_SC_PARAMS,
)


AROWS = EPT // D
NV = EPT // D

_SC_REG_PARAMS = pltpu.CompilerParams(use_tc_tiling_on_sc=False,
                                      needs_layout_passes=False)


ARPT = AROWS // NS
IDN = AROWS // 128


def _reg_reduce_epilogue(cid, sid, acc2d, acc_sh, idn_v, zeros_hbm, out_hbm):
    plsc.subcore_barrier()
    for j in range(IDN):
        pltpu.sync_copy(acc2d.at[pl.ds(j * 128, 128)],
                        acc_sh.at[idn_v.at[j]], add=True)
    plsc.subcore_barrier()
    pltpu.sync_copy(acc_sh.at[pl.ds(sid * ARPT, ARPT)],
                    out_hbm.at[cid, pl.ds(sid * ARPT, ARPT)])


NFR = N_PAD // D


def _idx_slice(idx_v, k):
    return idx_v[lax.shift_right_logical(k, 3), pl.ds((k & 7) * D, D)]


def _sc_deg_body(epad_hbm, zeros_hbm, iota_hbm, outr_hbm,
                 didx_v, idn_v, acc2d, accv, outr_v, acc_sh):
    cid = lax.axis_index("c")
    sid = lax.axis_index("s")
    wid = sid * NC + cid
    pltpu.sync_copy(epad_hbm.at[1, wid], didx_v)
    pltpu.sync_copy(iota_hbm, idn_v)
    pltpu.sync_copy(zeros_hbm.at[pl.ds(0, AROWS)], acc2d)
    pltpu.sync_copy(zeros_hbm.at[pl.ds(sid * ARPT, ARPT)],
                    acc_sh.at[pl.ds(sid * ARPT, ARPT)])
    ones16 = jnp.ones((D,), jnp.float32)

    def step(k, carry):
        d = _idx_slice(didx_v, k)
        plsc.addupdate_scatter(
            acc2d, [jnp.right_shift(d, 4), jnp.bitwise_and(d, 15)], ones16)
        return carry

    lax.fori_loop(0, NV, step, 0)
    plsc.subcore_barrier()
    for j in range(IDN):
        pltpu.sync_copy(acc2d.at[pl.ds(j * 128, 128)],
                        acc_sh.at[idn_v.at[j]], add=True)
    plsc.subcore_barrier()
    pltpu.sync_copy(acc_sh, accv)
    gbase = sid * ARPT

    def bstep(g, carry):
        v = accv[gbase + g, :]
        for l in range(D):
            outr_v[g * D + l, :] = jnp.broadcast_to(v[l], (D,))
        return carry

    lax.fori_loop(0, ARPT, bstep, 0)
    pltpu.sync_copy(outr_v, outr_hbm.at[cid, pl.ds(sid * ARPT * D, ARPT * D)])


_sc_deg = pl.kernel(
    _sc_deg_body,
    out_type=jax.ShapeDtypeStruct((NC, AROWS * D, D), jnp.float32),
    mesh=_MESH,
    scratch_types=[
        pltpu.VMEM((KJ, 128), jnp.int32),
        pltpu.VMEM((IDN, 128), jnp.int32),
        pltpu.VMEM((AROWS, D), jnp.float32),
        pltpu.VMEM((AROWS, D), jnp.float32),
        pltpu.VMEM((ARPT * D, D), jnp.float32),
        pltpu.VMEM_SHARED((AROWS, D), jnp.float32),
    ],
    compiler_params=_SC_REG_PARAMS,
)


def _tc_pre_body(x_ref, w1_ref, degr_ref, xs_ref, dis_ref):
    dis = lax.rsqrt(1.0 + (degr_ref[0] + degr_ref[1])[:N_PAD])
    xw = jnp.dot(x_ref[...], w1_ref[...], preferred_element_type=jnp.float32)
    xs_ref[...] = xw * dis
    dis_ref[...] = dis


_tc_pre = pl.pallas_call(
    _tc_pre_body,
    out_shape=(
        jax.ShapeDtypeStruct((N_PAD, D), jnp.float32),
        jax.ShapeDtypeStruct((N_PAD, D), jnp.float32),
    ),
)


def _tc_mid_body(acc1p_ref, xs_ref, dis_ref, b1_ref, w2b_ref, ysb_ref):
    acc = acc1p_ref[0] + acc1p_ref[1] + xs_ref[...]
    h = jnp.maximum(dis_ref[...] * acc + b1_ref[...], 0.0)
    y = jnp.dot(h, w2b_ref[...], preferred_element_type=jnp.float32)
    ysb_ref[...] = dis_ref[...] * y


_tc_mid = pl.pallas_call(
    _tc_mid_body,
    out_shape=jax.ShapeDtypeStruct((N_PAD, D), jnp.float32),
)


def _tc_out_body(acc2p_ref, ysb_ref, dis_ref, b2_ref, out_ref):
    acc = acc2p_ref[0] + acc2p_ref[1] + ysb_ref[...]
    out_ref[...] = dis_ref[...] * acc + b2_ref[...]


_tc_out = pl.pallas_call(
    _tc_out_body,
    out_shape=jax.ShapeDtypeStruct((N_PAD, D), jnp.float32),
)


@jax.jit
def kernel(x, edge_index, W1, b1, W2, b2):
    epad = jnp.pad(edge_index.astype(jnp.int32), ((0, 0), (0, E_PAD - E)),
                   constant_values=N).reshape(2, NW, KJ, 128)
    x_pad = jnp.pad(x, ((0, N_PAD - N), (0, 0)))
    zeros16 = jnp.zeros((N_PAD, D), jnp.float32)
    iota5 = jnp.arange(AROWS, dtype=jnp.int32).reshape(IDN, 128)

    degr = _sc_deg(epad, zeros16, iota5)
    xs, dis = _tc_pre(x_pad, W1, degr)
    acc1p = _sc_agg(xs, epad, zeros16)
    ysb = _tc_mid(acc1p, xs, dis, b1.reshape(1, D),
                  jnp.broadcast_to(W2, (D, D)))
    acc2p = _sc_agg(ysb, epad, zeros16)
    out16 = _tc_out(acc2p, ysb, dis, b2.reshape(1, 1))
    return out16[:N, :1]


# speedup vs baseline: 1.4170x; 1.4170x over previous
import jax
import jax.numpy as jnp
from jax import lax
from jax.experimental import pallas as pl
from jax.experimental.pallas import tpu as pltpu
from jax.experimental.pallas import tpu_sc as plsc

N = 10000
D = 16
D_FEAT = 128
NC, NS = 2, 16
NW = NC * NS
N_PAD = 10112
RPT = N_PAD // NS
E = 320000
EPT = 10240
E_PAD = NW * EPT
KJ = EPT // 128
CH = 10
NCHUNK = KJ // CH
HALF = NCHUNK // 2

_MESH = plsc.VectorSubcoreMesh(core_axis_name="c", subcore_axis_name="s")
_SC_PARAMS = pltpu.CompilerParams(use_tc_tiling_on_sc=False)


def _fire_gathers(vals_ref, sidx_v, buf, gsem, c):
    return [
        pltpu.async_copy(vals_ref.at[sidx_v.at[c * CH + j]],
                         buf.at[pl.ds(j * 128, 128)], gsem)
        for j in range(CH)
    ]


def _wait_gathers(vals_ref, sidx_v, buf, gsem):
    for j in range(CH):
        pltpu.make_async_copy(vals_ref.at[sidx_v.at[j]],
                              buf.at[pl.ds(j * 128, 128)], gsem).wait()


def _scatter_chunk(acc_sh, didx_v, buf, ssem, c):
    copies = [
        pltpu.async_copy(buf.at[pl.ds(j * 128, 128)],
                         acc_sh.at[didx_v.at[c * CH + j]], ssem, add=True)
        for j in range(CH)
    ]
    for cp in copies:
        cp.wait()


def _agg_pipeline(vals_ref, sidx_v, didx_v, rows_a, rows_b, acc_sh, gsem, ssem):
    _fire_gathers(vals_ref, sidx_v, rows_a, gsem, 0)

    def it(i, carry):
        c0 = 2 * i
        _wait_gathers(vals_ref, sidx_v, rows_a, gsem)
        _fire_gathers(vals_ref, sidx_v, rows_b, gsem, c0 + 1)
        _scatter_chunk(acc_sh, didx_v, rows_a, ssem, c0)
        _wait_gathers(vals_ref, sidx_v, rows_b, gsem)
        _fire_gathers(vals_ref, sidx_v, rows_a, gsem, c0 + 2)
        _scatter_chunk(acc_sh, didx_v, rows_b, ssem, c0 + 1)
        return carry

    lax.fori_loop(0, HALF - 1, it, 0)
    c0 = NCHUNK - 2
    _wait_gathers(vals_ref, sidx_v, rows_a, gsem)
    _fire_gathers(vals_ref, sidx_v, rows_b, gsem, c0 + 1)
    _scatter_chunk(acc_sh, didx_v, rows_a, ssem, c0)
    _wait_gathers(vals_ref, sidx_v, rows_b, gsem)
    _scatter_chunk(acc_sh, didx_v, rows_b, ssem, c0 + 1)


def _sc_agg_body(vals_hbm, epad_hbm, zeros_hbm, out_hbm,
                 sidx_v, didx_v, rows_a, rows_b, acc_sh, vals_sh, gsem, ssem):
    cid = lax.axis_index("c")
    sid = lax.axis_index("s")
    wid = sid * NC + cid
    pltpu.sync_copy(epad_hbm.at[0, wid], sidx_v)
    pltpu.sync_copy(epad_hbm.at[1, wid], didx_v)
    pltpu.sync_copy(vals_hbm.at[pl.ds(sid * RPT, RPT)],
                    vals_sh.at[pl.ds(sid * RPT, RPT)])
    pltpu.sync_copy(zeros_hbm.at[pl.ds(sid * RPT, RPT)],
                    acc_sh.at[pl.ds(sid * RPT, RPT)])
    plsc.subcore_barrier()
    _agg_pipeline(vals_sh, sidx_v, didx_v, rows_a, rows_b, acc_sh, gsem, ssem)
    plsc.subcore_barrier()
    pltpu.sync_copy(acc_sh.at[pl.ds(sid * RPT, RPT)],
                    out_hbm.at[cid, pl.ds(sid * RPT, RPT)])


_sc_agg = pl.kernel(
    _sc_agg_body,
    out_type=jax.ShapeDtypeStruct((NC, N_PAD, D), jnp.float32),
    mesh=_MESH,
    scratch_types=[
        pltpu.VMEM((KJ, 128), jnp.int32),
        pltpu.VMEM((KJ, 128), jnp.int32),
        pltpu.VMEM((CH * 128, D), jnp.float32),
        pltpu.VMEM((CH * 128, D), jnp.float32),
        pltpu.VMEM_SHARED((N_PAD, D), jnp.float32),
        pltpu.VMEM_SHARED((N_PAD, D), jnp.float32),
        pltpu.SemaphoreType.DMA,
        pltpu.SemaphoreType.DMA,
    ],
    compiler_params=_SC_PARAMS,
)


AROWS = EPT // D
NV = EPT // D

_SC_REG_PARAMS = pltpu.CompilerParams(use_tc_tiling_on_sc=False,
                                      needs_layout_passes=False)


ARPT = AROWS // NS
IDN = AROWS // 128


def _reg_reduce_epilogue(cid, sid, acc2d, acc_sh, idn_v, zeros_hbm, out_hbm):
    plsc.subcore_barrier()
    for j in range(IDN):
        pltpu.sync_copy(acc2d.at[pl.ds(j * 128, 128)],
                        acc_sh.at[idn_v.at[j]], add=True)
    plsc.subcore_barrier()
    pltpu.sync_copy(acc_sh.at[pl.ds(sid * ARPT, ARPT)],
                    out_hbm.at[cid, pl.ds(sid * ARPT, ARPT)])


NFR = N_PAD // D


def _idx_slice(idx_v, k):
    return idx_v[lax.shift_right_logical(k, 3), pl.ds((k & 7) * D, D)]


def _sc_deg_body(epad_hbm, zeros_hbm, iota_hbm, outr_hbm,
                 didx_v, idn_v, acc2d, accv, outr_v, acc_sh):
    cid = lax.axis_index("c")
    sid = lax.axis_index("s")
    wid = sid * NC + cid
    pltpu.sync_copy(epad_hbm.at[1, wid], didx_v)
    pltpu.sync_copy(iota_hbm, idn_v)
    pltpu.sync_copy(zeros_hbm.at[pl.ds(0, AROWS)], acc2d)
    pltpu.sync_copy(zeros_hbm.at[pl.ds(sid * ARPT, ARPT)],
                    acc_sh.at[pl.ds(sid * ARPT, ARPT)])
    ones16 = jnp.ones((D,), jnp.float32)

    def step(k, carry):
        d = _idx_slice(didx_v, k)
        plsc.addupdate_scatter(
            acc2d, [jnp.right_shift(d, 4), jnp.bitwise_and(d, 15)], ones16)
        return carry

    lax.fori_loop(0, NV, step, 0)
    plsc.subcore_barrier()
    for j in range(IDN):
        pltpu.sync_copy(acc2d.at[pl.ds(j * 128, 128)],
                        acc_sh.at[idn_v.at[j]], add=True)
    plsc.subcore_barrier()
    pltpu.sync_copy(acc_sh, accv)
    gbase = sid * ARPT

    def bstep(g, carry):
        v = accv[gbase + g, :]
        for l in range(D):
            outr_v[2 * g + (l // 8), pl.ds((l % 8) * D, D)] = (
                jnp.broadcast_to(v[l], (D,)))
        return carry

    lax.fori_loop(0, ARPT, bstep, 0)
    pltpu.sync_copy(outr_v, outr_hbm.at[cid, pl.ds(sid * 2 * ARPT, 2 * ARPT)])


_sc_deg = pl.kernel(
    _sc_deg_body,
    out_type=jax.ShapeDtypeStruct((NC, AROWS * D // 8, 128), jnp.float32),
    mesh=_MESH,
    scratch_types=[
        pltpu.VMEM((KJ, 128), jnp.int32),
        pltpu.VMEM((IDN, 128), jnp.int32),
        pltpu.VMEM((AROWS, D), jnp.float32),
        pltpu.VMEM((AROWS, D), jnp.float32),
        pltpu.VMEM((2 * ARPT, 128), jnp.float32),
        pltpu.VMEM_SHARED((AROWS, D), jnp.float32),
    ],
    compiler_params=_SC_REG_PARAMS,
)


NPKR = N_PAD // 8
NPKA = AROWS * 2


def _tc_pre_body(x_ref, w1_ref, degr_ref, xs_ref, dis_ref):
    dis = lax.rsqrt(1.0 + (degr_ref[0] + degr_ref[1])[:NPKR])
    xw = jnp.dot(x_ref[...], w1_ref[...], preferred_element_type=jnp.float32)
    xs_ref[...] = xw * dis
    dis_ref[...] = dis


_tc_pre = pl.pallas_call(
    _tc_pre_body,
    out_shape=(
        jax.ShapeDtypeStruct((NPKR, 128), jnp.float32),
        jax.ShapeDtypeStruct((NPKR, 128), jnp.float32),
    ),
)


def _tc_mid_body(acc1p_ref, xs_ref, dis_ref, b1_ref, w2b_ref, ysb_ref):
    acc = acc1p_ref[0] + acc1p_ref[1] + xs_ref[...]
    h = jnp.maximum(dis_ref[...] * acc + b1_ref[...], 0.0)
    y = jnp.dot(h, w2b_ref[...], preferred_element_type=jnp.float32)
    ysb_ref[...] = dis_ref[...] * y


_tc_mid = pl.pallas_call(
    _tc_mid_body,
    out_shape=jax.ShapeDtypeStruct((NPKR, 128), jnp.float32),
)


def _tc_out_body(acc2p_ref, ysb_ref, dis_ref, b2_ref, out_ref):
    acc = acc2p_ref[0] + acc2p_ref[1] + ysb_ref[...]
    out_ref[...] = dis_ref[...] * acc + b2_ref[...]


_tc_out = pl.pallas_call(
    _tc_out_body,
    out_shape=jax.ShapeDtypeStruct((NPKR, 128), jnp.float32),
)


@jax.jit
def kernel(x, edge_index, W1, b1, W2, b2):
    pk = jnp.float32
    epad = jnp.pad(edge_index.astype(jnp.int32), ((0, 0), (0, E_PAD - E)),
                   constant_values=N).reshape(2, NW, KJ, 128)
    x_pad = jnp.pad(x, ((0, N_PAD - N), (0, 0))).reshape(NPKR, 8 * D_FEAT)
    zeros16 = jnp.zeros((N_PAD, D), pk)
    iota5 = jnp.arange(AROWS, dtype=jnp.int32).reshape(IDN, 128)
    eye8 = jnp.eye(8, dtype=pk)
    w1big = jnp.kron(eye8, W1)
    w2big = jnp.kron(eye8, jnp.broadcast_to(W2, (D, D)))
    b1v = jnp.tile(b1, 8).reshape(1, 128)

    degrv = _sc_deg(epad, zeros16, iota5)
    xsv, disv = _tc_pre(x_pad, w1big, degrv)
    acc1p = _sc_agg(xsv.reshape(N_PAD, D), epad, zeros16)
    ysbv = _tc_mid(acc1p.reshape(NC, NPKR, 128), xsv, disv, b1v, w2big)
    acc2p = _sc_agg(ysbv.reshape(N_PAD, D), epad, zeros16)
    outv = _tc_out(acc2p.reshape(NC, NPKR, 128), ysbv, disv, b2.reshape(1, 1))
    return outv.reshape(N_PAD, D)[:N, :1]
